# baseline (speedup 1.0000x reference)
import functools

import jax
import jax.numpy as jnp
from jax import lax
from jax.experimental import pallas as pl
from jax.experimental.pallas import tpu as pltpu
from jax.experimental.pallas import tpu_sc as plsc

BATCH = 16384
D = 128
NC = 2
NS = 16
NW = NC * NS
BPW = BATCH // NW
CH = 32
NCHUNK = BPW // CH


def _mf_body(u_hbm, i_hbm, eu_hbm, ei_hbm, out_hbm,
             idx_u, idx_i, rows_u3, rows_i3,
             out_v, stag,
             sem_iu, sem_ii, sem_u0, sem_i0, sem_u1, sem_i1):
    wid = lax.axis_index("s") * NC + lax.axis_index("c")
    base = wid * BPW

    cps = []
    for c in range(NCHUNK):
        cps.append(pltpu.async_copy(
            u_hbm.at[pl.ds(base + c * CH, CH)], idx_u.at[c], sem_iu))
        cps.append(pltpu.async_copy(
            i_hbm.at[pl.ds(base + c * CH, CH)], idx_i.at[c], sem_ii))
    for cp in cps:
        cp.wait()

    sems = ((sem_u0, sem_i0), (sem_u1, sem_i1))

    def fire(c, p):
        su, si = sems[p]
        pltpu.async_copy(eu_hbm.at[idx_u.at[c]], rows_u3.at[p], su)
        pltpu.async_copy(ei_hbm.at[idx_i.at[c]], rows_i3.at[p], si)

    def drain(p):
        su, si = sems[p]
        pltpu.make_async_copy(eu_hbm.at[idx_u.at[0]], rows_u3.at[p], su).wait()
        pltpu.make_async_copy(ei_hbm.at[idx_i.at[0]], rows_i3.at[p], si).wait()

    col_base = lax.iota(jnp.int32, 16) * 17

    def dot_chunk(p, c):
        def g_body(g, _):
            for rr in range(16):
                r = g * 16 + rr
                acc = rows_u3[p, r, pl.ds(0, 16)] * rows_i3[p, r, pl.ds(0, 16)]
                for k in range(1, D // 16):
                    acc = acc + (rows_u3[p, r, pl.ds(16 * k, 16)]
                                 * rows_i3[p, r, pl.ds(16 * k, 16)])
                stag[pl.ds(rr * 17, 16)] = acc
            colsum = plsc.load_gather(stag, [col_base])
            for j in range(1, 16):
                colsum = colsum + plsc.load_gather(stag, [col_base + j])
            out_v[pl.ds(c * CH + g * 16, 16)] = colsum
            return 0

        lax.fori_loop(0, CH // 16, g_body, 0)

    fire(0, 0)
    fire(1, 1)

    def c_body(c, _):
        p = lax.rem(c, 2)

        @pl.when(p == 0)
        def _():
            drain(0)

        @pl.when(p == 1)
        def _():
            drain(1)

        dot_chunk(p, c)

        @pl.when(jnp.logical_and(c + 2 < NCHUNK, p == 0))
        def _():
            fire(c + 2, 0)

        @pl.when(jnp.logical_and(c + 2 < NCHUNK, p == 1))
        def _():
            fire(c + 2, 1)

        return 0

    lax.fori_loop(0, NCHUNK, c_body, 0)

    pltpu.sync_copy(out_v, out_hbm.at[pl.ds(base, BPW)])


@jax.jit
def _mf(u, i, emb_u, emb_i):
    run = pl.kernel(
        _mf_body,
        out_type=jax.ShapeDtypeStruct((BATCH,), jnp.float32),
        mesh=plsc.VectorSubcoreMesh(core_axis_name="c", subcore_axis_name="s"),
        compiler_params=pltpu.CompilerParams(needs_layout_passes=False),
        scratch_types=[
            pltpu.VMEM((NCHUNK, CH), jnp.int32),
            pltpu.VMEM((NCHUNK, CH), jnp.int32),
            pltpu.VMEM((2, CH, D), jnp.float32),
            pltpu.VMEM((2, CH, D), jnp.float32),
            pltpu.VMEM((BPW,), jnp.float32),
            pltpu.VMEM((16 * 17,), jnp.float32),
            pltpu.SemaphoreType.DMA,
            pltpu.SemaphoreType.DMA,
            pltpu.SemaphoreType.DMA,
            pltpu.SemaphoreType.DMA,
            pltpu.SemaphoreType.DMA,
            pltpu.SemaphoreType.DMA,
        ],
    )
    return run(u, i, emb_u, emb_i)


def kernel(u, i, emb_u, emb_i):
    return _mf(u.astype(jnp.int32), i.astype(jnp.int32), emb_u, emb_i)


# speedup vs baseline: 1.0638x; 1.0638x over previous
import functools

import jax
import jax.numpy as jnp
from jax import lax
from jax.experimental import pallas as pl
from jax.experimental.pallas import tpu as pltpu
from jax.experimental.pallas import tpu_sc as plsc

BATCH = 16384
D = 128
NC = 2
NS = 16
NW = NC * NS
BPW = BATCH // NW
CH = 64
NCHUNK = BPW // CH


def _mf_body(u_hbm, i_hbm, eu_hbm, ei_hbm, out_hbm,
             idx_u, idx_i, rows_u3, rows_i3,
             out_v, stag,
             sem_iu, sem_ii, sem_u0, sem_i0, sem_u1, sem_i1, sem_u2, sem_i2):
    wid = lax.axis_index("s") * NC + lax.axis_index("c")
    base = wid * BPW

    cps = []
    for c in range(NCHUNK):
        cps.append(pltpu.async_copy(
            u_hbm.at[pl.ds(base + c * CH, CH)], idx_u.at[c], sem_iu))
        cps.append(pltpu.async_copy(
            i_hbm.at[pl.ds(base + c * CH, CH)], idx_i.at[c], sem_ii))
    for cp in cps:
        cp.wait()

    sems = ((sem_u0, sem_i0), (sem_u1, sem_i1), (sem_u2, sem_i2))

    def fire(c, p):
        su, si = sems[p]
        pltpu.async_copy(eu_hbm.at[idx_u.at[c]], rows_u3.at[p], su)
        pltpu.async_copy(ei_hbm.at[idx_i.at[c]], rows_i3.at[p], si)

    def drain(p):
        su, si = sems[p]
        pltpu.make_async_copy(eu_hbm.at[idx_u.at[0]], rows_u3.at[p], su).wait()
        pltpu.make_async_copy(ei_hbm.at[idx_i.at[0]], rows_i3.at[p], si).wait()

    col_base = lax.iota(jnp.int32, 16) * 17

    def dot_chunk(p, c):
        def g_body(g, _):
            for rr in range(16):
                r = g * 16 + rr
                acc = rows_u3[p, r, pl.ds(0, 16)] * rows_i3[p, r, pl.ds(0, 16)]
                for k in range(1, D // 16):
                    acc = acc + (rows_u3[p, r, pl.ds(16 * k, 16)]
                                 * rows_i3[p, r, pl.ds(16 * k, 16)])
                stag[pl.ds(rr * 17, 16)] = acc
            colsum = plsc.load_gather(stag, [col_base])
            for j in range(1, 16):
                colsum = colsum + plsc.load_gather(stag, [col_base + j])
            out_v[pl.ds(c * CH + g * 16, 16)] = colsum
            return 0

        lax.fori_loop(0, CH // 16, g_body, 0)

    fire(0, 0)
    fire(1, 1)
    fire(2, 2)

    def c_body(c, _):
        p = lax.rem(c, 3)

        for q in range(3):
            @pl.when(p == q)
            def _(q=q):
                drain(q)

        dot_chunk(p, c)

        for q in range(3):
            @pl.when(jnp.logical_and(c + 3 < NCHUNK, p == q))
            def _(q=q):
                fire(c + 3, q)

        return 0

    lax.fori_loop(0, NCHUNK, c_body, 0)

    pltpu.sync_copy(out_v, out_hbm.at[pl.ds(base, BPW)])


@jax.jit
def _mf(u, i, emb_u, emb_i):
    run = pl.kernel(
        _mf_body,
        out_type=jax.ShapeDtypeStruct((BATCH,), jnp.float32),
        mesh=plsc.VectorSubcoreMesh(core_axis_name="c", subcore_axis_name="s"),
        compiler_params=pltpu.CompilerParams(needs_layout_passes=False),
        scratch_types=[
            pltpu.VMEM((NCHUNK, CH), jnp.int32),
            pltpu.VMEM((NCHUNK, CH), jnp.int32),
            pltpu.VMEM((3, CH, D), jnp.float32),
            pltpu.VMEM((3, CH, D), jnp.float32),
            pltpu.VMEM((BPW,), jnp.float32),
            pltpu.VMEM((16 * 17,), jnp.float32),
            pltpu.SemaphoreType.DMA,
            pltpu.SemaphoreType.DMA,
            pltpu.SemaphoreType.DMA,
            pltpu.SemaphoreType.DMA,
            pltpu.SemaphoreType.DMA,
            pltpu.SemaphoreType.DMA,
            pltpu.SemaphoreType.DMA,
            pltpu.SemaphoreType.DMA,
        ],
    )
    return run(u, i, emb_u, emb_i)


def kernel(u, i, emb_u, emb_i):
    return _mf(u.astype(jnp.int32), i.astype(jnp.int32), emb_u, emb_i)


# speedup vs baseline: 1.0686x; 1.0045x over previous
import functools

import jax
import jax.numpy as jnp
from jax import lax
from jax.experimental import pallas as pl
from jax.experimental.pallas import tpu as pltpu
from jax.experimental.pallas import tpu_sc as plsc

BATCH = 16384
D = 128
NC = 2
NS = 16
NW = NC * NS
BPW = BATCH // NW
CH = 64
NCHUNK = BPW // CH


def _mf_body(u_hbm, i_hbm, eu_hbm, ei_hbm, out_hbm,
             idx_u, idx_i, rows_u3, rows_i3,
             out_v, stag,
             sem_iu, sem_ii, sem_u0, sem_i0, sem_u1, sem_i1, sem_u2, sem_i2):
    wid = lax.axis_index("s") * NC + lax.axis_index("c")
    base = wid * BPW

    cps = []
    for c in range(NCHUNK):
        cps.append(pltpu.async_copy(
            u_hbm.at[pl.ds(base + c * CH, CH)], idx_u.at[c], sem_iu))
        cps.append(pltpu.async_copy(
            i_hbm.at[pl.ds(base + c * CH, CH)], idx_i.at[c], sem_ii))
    for cp in cps:
        cp.wait()

    sems = ((sem_u0, sem_i0), (sem_u1, sem_i1), (sem_u2, sem_i2))

    def fire(c, p):
        su, si = sems[p]
        pltpu.async_copy(eu_hbm.at[idx_u.at[c]], rows_u3.at[p], su)
        pltpu.async_copy(ei_hbm.at[idx_i.at[c]], rows_i3.at[p], si)

    def drain(p):
        su, si = sems[p]
        pltpu.make_async_copy(eu_hbm.at[idx_u.at[0]], rows_u3.at[p], su).wait()
        pltpu.make_async_copy(ei_hbm.at[idx_i.at[0]], rows_i3.at[p], si).wait()

    col_base = lax.iota(jnp.int32, 16) * 17

    def dot_chunk(p, c):
        def g_body(g, _):
            for rr in range(16):
                r = g * 16 + rr
                acc = rows_u3[p, r, pl.ds(0, 16)] * rows_i3[p, r, pl.ds(0, 16)]
                for k in range(1, D // 16):
                    acc = acc + (rows_u3[p, r, pl.ds(16 * k, 16)]
                                 * rows_i3[p, r, pl.ds(16 * k, 16)])
                stag[pl.ds(rr * 17, 16)] = acc
            cols = [plsc.load_gather(stag, [col_base + j]) for j in range(16)]
            while len(cols) > 1:
                cols = [cols[i] + cols[i + 1] for i in range(0, len(cols), 2)]
            out_v[pl.ds(c * CH + g * 16, 16)] = cols[0]
            return 0

        lax.fori_loop(0, CH // 16, g_body, 0)

    fire(0, 0)
    fire(1, 1)
    fire(2, 2)

    def c_body(c, _):
        p = lax.rem(c, 3)

        for q in range(3):
            @pl.when(p == q)
            def _(q=q):
                drain(q)

        dot_chunk(p, c)

        for q in range(3):
            @pl.when(jnp.logical_and(c + 3 < NCHUNK, p == q))
            def _(q=q):
                fire(c + 3, q)

        return 0

    lax.fori_loop(0, NCHUNK, c_body, 0)

    pltpu.sync_copy(out_v, out_hbm.at[pl.ds(base, BPW)])


@jax.jit
def _mf(u, i, emb_u, emb_i):
    run = pl.kernel(
        _mf_body,
        out_type=jax.ShapeDtypeStruct((BATCH,), jnp.float32),
        mesh=plsc.VectorSubcoreMesh(core_axis_name="c", subcore_axis_name="s"),
        compiler_params=pltpu.CompilerParams(needs_layout_passes=False),
        scratch_types=[
            pltpu.VMEM((NCHUNK, CH), jnp.int32),
            pltpu.VMEM((NCHUNK, CH), jnp.int32),
            pltpu.VMEM((3, CH, D), jnp.float32),
            pltpu.VMEM((3, CH, D), jnp.float32),
            pltpu.VMEM((BPW,), jnp.float32),
            pltpu.VMEM((16 * 17,), jnp.float32),
            pltpu.SemaphoreType.DMA,
            pltpu.SemaphoreType.DMA,
            pltpu.SemaphoreType.DMA,
            pltpu.SemaphoreType.DMA,
            pltpu.SemaphoreType.DMA,
            pltpu.SemaphoreType.DMA,
            pltpu.SemaphoreType.DMA,
            pltpu.SemaphoreType.DMA,
        ],
    )
    return run(u, i, emb_u, emb_i)


def kernel(u, i, emb_u, emb_i):
    return _mf(u.astype(jnp.int32), i.astype(jnp.int32), emb_u, emb_i)
